# baseline (speedup 1.0000x reference)
import functools

import jax
import jax.numpy as jnp
import numpy as np
from jax import lax
from jax.experimental import pallas as pl
from jax.experimental.pallas import tpu as pltpu
from jax.experimental.pallas import tpu_sc as plsc

RHO_NUM = 100
B = 16384
D = 11

_NUM_SUBCORES = 16
_ROWS_PER_TILE = B // _NUM_SUBCORES
_ROWS_PER_W = _ROWS_PER_TILE // 2
_GROUPS = _ROWS_PER_W // 16
_EQCHUNK = _ROWS_PER_TILE * D
_Z2_OFF = B * D
_DG_OFF = 2 * B * D


def _bucket_thresholds():
    c = np.float32(1.0 / RHO_NUM)
    t = np.array([np.float32(0.01 * i) for i in range(1, RHO_NUM)], np.float32)

    def bucket_id(s):
        r = np.sqrt(np.float32(s), dtype=np.float32)
        k = np.round(np.float32(r / c)).astype(np.float32)
        return int(np.sum(np.float32(k * c) >= t))

    out = np.empty(RHO_NUM - 1, np.float32)
    for j in range(1, RHO_NUM):
        lo, hi = 0, int(np.array(1e8, np.float32).view(np.uint32))
        while lo < hi:
            mid = (lo + hi) // 2
            if bucket_id(np.array(mid, np.uint32).view(np.float32)) >= j:
                hi = mid
            else:
                lo = mid + 1
        out[j - 1] = np.array(lo, np.uint32).view(np.float32)
    return out


_S_LIST = [float(v) for v in _bucket_thresholds()]


def _sc_body(buf_hbm, sat_hbm, z1v, z2v, idxv, maskv, valv, accv, eqv,
             eq_shared, sem):
    cid = lax.axis_index("c")
    sid = lax.axis_index("s")
    eqbase = sid * _EQCHUNK
    rbase = sid * _ROWS_PER_TILE + cid * _ROWS_PER_W
    lbase = cid * _ROWS_PER_W * D

    pltpu.sync_copy(buf_hbm.at[pl.ds(eqbase, _EQCHUNK)], z1v)
    pltpu.sync_copy(buf_hbm.at[pl.ds(_Z2_OFF + eqbase, _EQCHUNK)], z2v)

    lanes = lax.iota(jnp.int32, 16)

    def eqstep(k, acc):
        a = z1v[pl.ds(k * 16, 16)]
        b = z2v[pl.ds(k * 16, 16)]
        return jnp.where(a != b, 1.0, acc)

    neq_acc = lax.fori_loop(0, _EQCHUNK // 16, eqstep,
                            jnp.zeros((16,), jnp.float32))
    accv[...] = neq_acc
    pltpu.sync_copy(accv, eq_shared.at[sid])
    plsc.subcore_barrier()

    def group(g, _):
        ridx = lbase + (g * 16 + lanes) * D
        z1_0 = plsc.load_gather(z1v, [ridx])
        z2_0 = plsc.load_gather(z2v, [ridx])
        z1_x = plsc.load_gather(z1v, [ridx + (D - 2)])
        z2_x = plsc.load_gather(z2v, [ridx + (D - 2)])
        z1_y = plsc.load_gather(z1v, [ridx + (D - 1)])
        z2_y = plsc.load_gather(z2v, [ridx + (D - 1)])
        dx = z1_x - z2_x
        dy = z1_y - z2_y
        s = dx * dx + dy * dy
        mf = jnp.where((z1_0 > 0.0) & (z2_0 > 0.0), 1.0, 0.0)
        bid = jnp.zeros((16,), jnp.int32)
        for thr in _S_LIST:
            bid = bid + (s >= thr).astype(jnp.int32)
        gidx = _DG_OFF + (rbase + g * 16 + lanes) * RHO_NUM + bid
        idxv[pl.ds(g * 16, 16)] = gidx
        maskv[pl.ds(g * 16, 16)] = mf
        return 0

    lax.fori_loop(0, _GROUPS, group, 0)

    copies = [
        pltpu.async_copy(
            buf_hbm.at[idxv.at[pl.ds(i * 128, 128)]],
            valv.at[pl.ds(i * 128, 128)],
            sem,
        )
        for i in range(_ROWS_PER_W // 128)
    ]

    pltpu.sync_copy(eq_shared, eqv)
    ne = jnp.zeros((16,), jnp.float32)
    for i in range(_NUM_SUBCORES):
        ne = jnp.maximum(ne, eqv[i, :])
    any_ne = jnp.max(ne)
    gate = jnp.where(any_ne > 0.0, 1.0, 0.0)

    for c in copies:
        c.wait()

    for g in range(_GROUPS):
        sl = pl.ds(g * 16, 16)
        valv[sl] = valv[sl] * maskv[sl] * gate
    pltpu.sync_copy(valv, sat_hbm.at[pl.ds(rbase, _ROWS_PER_W)])


_sc_fn = functools.partial(
    pl.kernel,
    mesh=plsc.VectorSubcoreMesh(core_axis_name="c", subcore_axis_name="s"),
    compiler_params=pltpu.CompilerParams(needs_layout_passes=False),
    out_type=jax.ShapeDtypeStruct((B,), jnp.float32),
    scratch_types=[
        pltpu.VMEM((_EQCHUNK,), jnp.float32),
        pltpu.VMEM((_EQCHUNK,), jnp.float32),
        pltpu.VMEM((_ROWS_PER_W,), jnp.int32),
        pltpu.VMEM((_ROWS_PER_W,), jnp.float32),
        pltpu.VMEM((_ROWS_PER_W,), jnp.float32),
        pltpu.VMEM((16,), jnp.float32),
        pltpu.VMEM((_NUM_SUBCORES, 16), jnp.float32),
        pltpu.VMEM_SHARED((_NUM_SUBCORES, 16), jnp.float32),
        pltpu.SemaphoreType.DMA,
    ],
)(_sc_body)


def kernel(z_1, z_2, dist_grade, img, given_param):
    buf = jnp.concatenate(
        [z_1.reshape(-1), z_2.reshape(-1), dist_grade.reshape(-1)]
    )
    return _sc_fn(buf)


# speedup vs baseline: 1.0640x; 1.0640x over previous
import functools

import jax
import jax.numpy as jnp
import numpy as np
from jax import lax
from jax.experimental import pallas as pl
from jax.experimental.pallas import tpu as pltpu
from jax.experimental.pallas import tpu_sc as plsc

RHO_NUM = 100
B = 16384
D = 11

_NUM_SUBCORES = 16
_ROWS_PER_TILE = B // _NUM_SUBCORES
_ROWS_PER_W = _ROWS_PER_TILE // 2
_GROUPS = _ROWS_PER_W // 16
_EQCHUNK = _ROWS_PER_TILE * D
_DGW = 128


def _bucket_thresholds():
    c = np.float32(1.0 / RHO_NUM)
    t = np.array([np.float32(0.01 * i) for i in range(1, RHO_NUM)], np.float32)

    def bucket_id(s):
        r = np.sqrt(np.float32(s), dtype=np.float32)
        k = np.round(np.float32(r / c)).astype(np.float32)
        return int(np.sum(np.float32(k * c) >= t))

    out = np.empty(RHO_NUM - 1, np.float32)
    for j in range(1, RHO_NUM):
        lo, hi = 0, int(np.array(1e8, np.float32).view(np.uint32))
        while lo < hi:
            mid = (lo + hi) // 2
            if bucket_id(np.array(mid, np.uint32).view(np.float32)) >= j:
                hi = mid
            else:
                lo = mid + 1
        out[j - 1] = np.array(lo, np.uint32).view(np.float32)
    return out


_S_LIST = [float(v) for v in _bucket_thresholds()]


def _sc_body(z1_hbm, z2_hbm, dg_hbm, sat_hbm, z1v, z2v, idxv, maskv, valv,
             accv, eqv, eq_shared, sem):
    cid = lax.axis_index("c")
    sid = lax.axis_index("s")
    eqbase = sid * _EQCHUNK
    rbase = sid * _ROWS_PER_TILE + cid * _ROWS_PER_W
    lbase = cid * _ROWS_PER_W * D

    pltpu.sync_copy(z1_hbm.at[pl.ds(eqbase, _EQCHUNK)], z1v)
    pltpu.sync_copy(z2_hbm.at[pl.ds(eqbase, _EQCHUNK)], z2v)

    lanes = lax.iota(jnp.int32, 16)

    def eqstep(k, acc):
        a = z1v[pl.ds(k * 16, 16)]
        b = z2v[pl.ds(k * 16, 16)]
        return jnp.where(a != b, 1.0, acc)

    neq_acc = lax.fori_loop(0, _EQCHUNK // 16, eqstep,
                            jnp.zeros((16,), jnp.float32))
    accv[...] = neq_acc
    pltpu.sync_copy(accv, eq_shared.at[sid])
    plsc.subcore_barrier()

    def group(g, _):
        ridx = lbase + (g * 16 + lanes) * D
        z1_0 = plsc.load_gather(z1v, [ridx])
        z2_0 = plsc.load_gather(z2v, [ridx])
        z1_x = plsc.load_gather(z1v, [ridx + (D - 2)])
        z2_x = plsc.load_gather(z2v, [ridx + (D - 2)])
        z1_y = plsc.load_gather(z1v, [ridx + (D - 1)])
        z2_y = plsc.load_gather(z2v, [ridx + (D - 1)])
        dx = z1_x - z2_x
        dy = z1_y - z2_y
        s = dx * dx + dy * dy
        mf = jnp.where((z1_0 > 0.0) & (z2_0 > 0.0), 1.0, 0.0)
        bid = jnp.zeros((16,), jnp.int32)
        for thr in _S_LIST:
            bid = bid + (s >= thr).astype(jnp.int32)
        gidx = (rbase + g * 16 + lanes) * _DGW + bid
        idxv[pl.ds(g * 16, 16)] = gidx
        maskv[pl.ds(g * 16, 16)] = mf
        return 0

    lax.fori_loop(0, _GROUPS, group, 0)

    copies = [
        pltpu.async_copy(
            dg_hbm.at[idxv.at[pl.ds(i * 128, 128)]],
            valv.at[pl.ds(i * 128, 128)],
            sem,
        )
        for i in range(_ROWS_PER_W // 128)
    ]

    pltpu.sync_copy(eq_shared, eqv)
    ne = jnp.zeros((16,), jnp.float32)
    for i in range(_NUM_SUBCORES):
        ne = jnp.maximum(ne, eqv[i, :])
    any_ne = jnp.max(ne)
    gate = jnp.where(any_ne > 0.0, 1.0, 0.0)

    for c in copies:
        c.wait()

    for g in range(_GROUPS):
        sl = pl.ds(g * 16, 16)
        valv[sl] = valv[sl] * maskv[sl] * gate
    pltpu.sync_copy(valv, sat_hbm.at[pl.ds(rbase, _ROWS_PER_W)])


_sc_fn = functools.partial(
    pl.kernel,
    mesh=plsc.VectorSubcoreMesh(core_axis_name="c", subcore_axis_name="s"),
    compiler_params=pltpu.CompilerParams(needs_layout_passes=False),
    out_type=jax.ShapeDtypeStruct((B,), jnp.float32),
    scratch_types=[
        pltpu.VMEM((_EQCHUNK,), jnp.float32),
        pltpu.VMEM((_EQCHUNK,), jnp.float32),
        pltpu.VMEM((_ROWS_PER_W,), jnp.int32),
        pltpu.VMEM((_ROWS_PER_W,), jnp.float32),
        pltpu.VMEM((_ROWS_PER_W,), jnp.float32),
        pltpu.VMEM((16,), jnp.float32),
        pltpu.VMEM((_NUM_SUBCORES, 16), jnp.float32),
        pltpu.VMEM_SHARED((_NUM_SUBCORES, 16), jnp.float32),
        pltpu.SemaphoreType.DMA,
    ],
)(_sc_body)


def _pad_body(dg_ref, out_ref):
    out_ref[:, : RHO_NUM] = dg_ref[...]
    out_ref[:, RHO_NUM:] = jnp.zeros((dg_ref.shape[0], _DGW - RHO_NUM),
                                     jnp.float32)


def _pad_dist_grade(dist_grade):
    blk = 2048
    return pl.pallas_call(
        _pad_body,
        grid=(B // blk,),
        in_specs=[pl.BlockSpec((blk, RHO_NUM), lambda i: (i, 0))],
        out_specs=pl.BlockSpec((blk, _DGW), lambda i: (i, 0)),
        out_shape=jax.ShapeDtypeStruct((B, _DGW), jnp.float32),
    )(dist_grade)


def kernel(z_1, z_2, dist_grade, img, given_param):
    dgf = _pad_dist_grade(dist_grade).reshape(-1)
    return _sc_fn(z_1.reshape(-1), z_2.reshape(-1), dgf)
